# baseline (speedup 1.0000x reference)
import jax
import jax.numpy as jnp
from jax.experimental import pallas as pl

N = 50000


def _leaky(x):
    return jnp.where(x >= 0, x, 0.01 * x)


def _l2norm(x):
    n = jnp.sqrt(jnp.sum(x * x, axis=-1, keepdims=True))
    return x / jnp.maximum(n, 1e-12)


def _gat(x, edge_index, W, b):
    row = edge_index[0]
    col = edge_index[1]
    xw = x @ W
    x_j = xw[row]
    x_i = xw[col]
    ip = jnp.sum(x_i * _leaky(x_j), axis=-1)
    deg = jnp.zeros((N,), dtype=x.dtype).at[row].add(1.0)
    d = deg[row]
    dis = jnp.where(d > 0, d ** -0.5, 0.0)
    gate = jax.nn.sigmoid(dis * ip)
    t = ip * gate
    m = jax.ops.segment_max(t, col, num_segments=N)
    m = jnp.where(jnp.isfinite(m), m, 0.0)
    e = jnp.exp(t - m[col])
    s = jax.ops.segment_sum(e, col, num_segments=N)
    attn = e / (s[col] + 1e-16)
    out = jax.ops.segment_sum(x_j * attn[:, None], col, num_segments=N)
    return _l2norm(out + b)


def _concat_kernel(a_ref, b_ref, c_ref, o_ref):
    o_ref[:, 0:64] = a_ref[...]
    o_ref[:, 64:128] = b_ref[...]
    o_ref[:, 128:192] = c_ref[...]


def kernel(features, preference, id_embedding, edge_index,
           mlp_w, mlp_b, conv1_w, conv1_b, lin1_w, lin1_b, g1_w, g1_b,
           conv2_w, conv2_b, lin2_w, lin2_b, g2_w, g2_b,
           conv3_w, conv3_b, lin3_w, lin3_b, g3_w, g3_b):
    mask = edge_index[0] != edge_index[1]
    ei = jnp.where(mask, edge_index, N)
    temp = jnp.tanh(features @ mlp_w + mlp_b)
    x = _l2norm(jnp.concatenate([preference, temp], axis=0))
    h = _leaky(_gat(x, ei, conv1_w, conv1_b))
    x_hat = _leaky(x @ lin1_w + lin1_b) + id_embedding
    x1 = _leaky(h @ g1_w + g1_b + x_hat)
    h = _leaky(_gat(x1, ei, conv2_w, conv2_b))
    x_hat = _leaky(x1 @ lin2_w + lin2_b) + id_embedding
    x2 = _leaky(h @ g2_w + g2_b + x_hat)
    h = _leaky(_gat(x2, ei, conv3_w, conv3_b))
    x_hat = _leaky(x2 @ lin3_w + lin3_b) + id_embedding
    x3 = _leaky(h @ g3_w + g3_b + x_hat)

    bm = 400
    out = pl.pallas_call(
        _concat_kernel,
        grid=(N // bm,),
        in_specs=[pl.BlockSpec((bm, 64), lambda i: (i, 0))] * 3,
        out_specs=pl.BlockSpec((bm, 192), lambda i: (i, 0)),
        out_shape=jax.ShapeDtypeStruct((N, 192), jnp.float32),
    )(x1, x2, x3)
    return out


# speedup vs baseline: 5.2643x; 5.2643x over previous
import functools

import jax
import jax.numpy as jnp
from jax import lax
from jax.experimental import pallas as pl
from jax.experimental.pallas import tpu as pltpu
from jax.experimental.pallas import tpu_sc as plsc

N = 50000
E = 800000
D = 64
NP = 51200
DUMMY = N
E_PAD = 819200
NW = 32
EPW = E_PAD // NW
CH = 128
SUP = 512
SUBS = SUP // CH
ROWS2 = E_PAD // CH
SLICE = NP // 16
SNP = 50176
SLICE_S = SNP // 16
BM = 400
BE = 2048

_f32 = jnp.float32
_i32 = jnp.int32


def _leaky(x):
    return jnp.where(x >= 0, x, 0.01 * x)


def _l2norm(x):
    n = jnp.sqrt(jnp.sum(x * x, axis=-1, keepdims=True))
    return x / jnp.maximum(n, 1e-12)


_MESH = plsc.VectorSubcoreMesh(core_axis_name="c", subcore_axis_name="s")
_SC_PARAMS = pltpu.CompilerParams(use_tc_tiling_on_sc=False)


@functools.partial(
    pl.kernel,
    mesh=_MESH,
    compiler_params=_SC_PARAMS,
    out_type=[
        jax.ShapeDtypeStruct((ROWS2, CH), _i32),
        jax.ShapeDtypeStruct((ROWS2, CH), _i32),
        jax.ShapeDtypeStruct((NP,), _f32),
        jax.ShapeDtypeStruct((NP,), _f32),
    ],
    scratch_types=[
        pltpu.VMEM((SUBS, CH), _i32),
        pltpu.VMEM((SUBS, CH), _i32),
        pltpu.VMEM((CH,), _f32),
        pltpu.VMEM((SLICE,), _f32),
        pltpu.VMEM_SHARED((NP,), _f32),
        pltpu.SemaphoreType.DMA,
    ],
)
def _sc_deg(row_hbm, col_hbm, rowm_hbm, colm_hbm, deg0_hbm, deg1_hbm,
            row_v, col_v, ones_v, zb_v, deg_sh, sem):
    c = lax.axis_index("c")
    s = lax.axis_index("s")
    wid = s * 2 + c

    def zb(k, carry):
        zb_v[pl.ds(k * 16, 16)] = jnp.zeros((16,), _f32)
        return carry

    lax.fori_loop(0, SLICE // 16, zb, 0)

    def ob(k, carry):
        ones_v[pl.ds(k * 16, 16)] = jnp.ones((16,), _f32)
        return carry

    lax.fori_loop(0, CH // 16, ob, 0)
    pltpu.sync_copy(zb_v, deg_sh.at[pl.ds(s * SLICE, SLICE)])
    plsc.subcore_barrier()

    r_per_w = EPW // CH
    base_r = wid * r_per_w

    def chunk(k, carry):
        r = base_r + k * SUBS
        pltpu.sync_copy(row_hbm.at[pl.ds(r, SUBS)], row_v)
        pltpu.sync_copy(col_hbm.at[pl.ds(r, SUBS)], col_v)
        for j in range(SUBS):
            for g in range(CH // 16):
                rv = row_v[j, pl.ds(g * 16, 16)]
                cv = col_v[j, pl.ds(g * 16, 16)]
                keep = rv != cv
                row_v[j, pl.ds(g * 16, 16)] = jnp.where(keep, rv, DUMMY)
                col_v[j, pl.ds(g * 16, 16)] = jnp.where(keep, cv, DUMMY)
        pltpu.sync_copy(row_v, rowm_hbm.at[pl.ds(r, SUBS)])
        pltpu.sync_copy(col_v, colm_hbm.at[pl.ds(r, SUBS)])
        cps = [pltpu.async_copy(ones_v, deg_sh.at[row_v.at[j]], sem, add=True)
               for j in range(SUBS)]
        for cp in cps:
            cp.wait()
        return carry

    lax.fori_loop(0, r_per_w // SUBS, chunk, 0)
    plsc.subcore_barrier()

    @pl.when(jnp.logical_and(s == 0, c == 0))
    def _():
        pltpu.sync_copy(deg_sh, deg0_hbm)

    @pl.when(jnp.logical_and(s == 0, c == 1))
    def _():
        pltpu.sync_copy(deg_sh, deg1_hbm)


@functools.partial(
    pl.kernel,
    mesh=_MESH,
    compiler_params=_SC_PARAMS,
    out_type=[
        jax.ShapeDtypeStruct((E_PAD, D), _f32),
        jax.ShapeDtypeStruct((E_PAD, D), _f32),
        jax.ShapeDtypeStruct((E_PAD,), _f32),
    ],
    scratch_types=[
        pltpu.VMEM((SUBS, CH), _i32),
        pltpu.VMEM((SUBS, CH), _i32),
        pltpu.VMEM((SUP, D), _f32),
        pltpu.VMEM((SUP, D), _f32),
        pltpu.VMEM((SUP,), _f32),
        pltpu.SemaphoreType.DMA,
        pltpu.SemaphoreType.DMA,
        pltpu.SemaphoreType.DMA,
        pltpu.SemaphoreType.DMA,
    ],
)
def _sc_gather(xw_hbm, dis_hbm, rowm_hbm, colm_hbm,
               xjg_hbm, xig_hbm, disg_hbm,
               rowm_v, colm_v, xj_v, xi_v, disg_v, sem_i, sem_a, sem_b, sem_c):
    c = lax.axis_index("c")
    s = lax.axis_index("s")
    wid = s * 2 + c
    r_per_w = EPW // CH
    base_r = wid * r_per_w

    def chunk(k, carry):
        r = base_r + k * SUBS
        b = r * CH
        ci1 = pltpu.async_copy(rowm_hbm.at[pl.ds(r, SUBS)], rowm_v, sem_i)
        ci2 = pltpu.async_copy(colm_hbm.at[pl.ds(r, SUBS)], colm_v, sem_i)
        ci1.wait()
        ci2.wait()
        cps = []
        for j in range(SUBS):
            cps.append(pltpu.async_copy(
                xw_hbm.at[rowm_v.at[j]],
                xj_v.at[pl.ds(j * CH, CH)], sem_a))
            cps.append(pltpu.async_copy(
                xw_hbm.at[colm_v.at[j]],
                xi_v.at[pl.ds(j * CH, CH)], sem_b))
            cps.append(pltpu.async_copy(
                dis_hbm.at[rowm_v.at[j]],
                disg_v.at[pl.ds(j * CH, CH)], sem_c))
        for cp in cps:
            cp.wait()
        pltpu.sync_copy(xj_v, xjg_hbm.at[pl.ds(b, SUP)])
        pltpu.sync_copy(xi_v, xig_hbm.at[pl.ds(b, SUP)])
        pltpu.sync_copy(disg_v, disg_hbm.at[pl.ds(b, SUP)])
        return carry

    lax.fori_loop(0, r_per_w // SUBS, chunk, 0)


@functools.partial(
    pl.kernel,
    mesh=_MESH,
    compiler_params=_SC_PARAMS,
    out_type=[
        jax.ShapeDtypeStruct((SNP,), _f32),
        jax.ShapeDtypeStruct((SNP,), _f32),
        jax.ShapeDtypeStruct((SNP, 32), _f32),
        jax.ShapeDtypeStruct((SNP, 32), _f32),
    ],
    scratch_types=[
        pltpu.VMEM((SUBS, CH), _i32),
        pltpu.VMEM((SUP,), _f32),
        pltpu.VMEM((SUP, 32), _f32),
        pltpu.VMEM((196, 32), _f32),
        pltpu.VMEM((SLICE_S,), _f32),
        pltpu.VMEM_SHARED((SNP,), _f32),
        pltpu.VMEM_SHARED((SNP, 32), _f32),
        pltpu.SemaphoreType.DMA,
        pltpu.SemaphoreType.DMA,
        pltpu.SemaphoreType.DMA,
    ],
)
def _sc_scatter(wxj_hbm, e_hbm, colm_hbm,
                s0_hbm, s1_hbm, agg0_hbm, agg1_hbm,
                colm_v, e_v, wxj_v, zrow_v, zb_v, s_sh, agg_sh,
                sem_l, sem_s, sem_e):
    c = lax.axis_index("c")
    s = lax.axis_index("s")

    def zb(k, carry):
        zb_v[pl.ds(k * 16, 16)] = jnp.zeros((16,), _f32)
        return carry

    lax.fori_loop(0, SLICE_S // 16, zb, 0)

    def zr(r, carry):
        zrow_v[r, pl.ds(0, 16)] = jnp.zeros((16,), _f32)
        zrow_v[r, pl.ds(16, 16)] = jnp.zeros((16,), _f32)
        return carry

    lax.fori_loop(0, 196, zr, 0)
    pltpu.sync_copy(zb_v, s_sh.at[pl.ds(s * SLICE_S, SLICE_S)])
    for q in range(16):
        pltpu.sync_copy(zrow_v, agg_sh.at[pl.ds(s * SLICE_S + q * 196, 196)])
    plsc.subcore_barrier()

    ept = E_PAD // 16
    r_per_t = ept // CH
    nch = r_per_t // SUBS

    def chunk(k, carry):
        r = s * r_per_t + k * SUBS
        b = r * CH
        cl1 = pltpu.async_copy(colm_hbm.at[pl.ds(r, SUBS)], colm_v, sem_l)
        cl2 = pltpu.async_copy(
            wxj_hbm.at[pl.ds(b, SUP), pl.ds(c * 32, 32)], wxj_v, sem_l)
        cl1.wait()
        cl2.wait()
        cps = [pltpu.async_copy(wxj_v.at[pl.ds(j * CH, CH)],
                                agg_sh.at[colm_v.at[j]], sem_s, add=True)
               for j in range(SUBS)]
        in_front = k < nch // 2
        do_e = jnp.logical_or(
            jnp.logical_and(c == 0, in_front),
            jnp.logical_and(c == 1, jnp.logical_not(in_front)))

        @pl.when(do_e)
        def _():
            pltpu.sync_copy(e_hbm.at[pl.ds(b, SUP)], e_v)
            ecps = [pltpu.async_copy(e_v.at[pl.ds(j * CH, CH)],
                                     s_sh.at[colm_v.at[j]], sem_e, add=True)
                    for j in range(SUBS)]
            for cp in ecps:
                cp.wait()

        for cp in cps:
            cp.wait()
        return carry

    lax.fori_loop(0, nch, chunk, 0)
    plsc.subcore_barrier()

    @pl.when(jnp.logical_and(s == 0, c == 0))
    def _():
        pltpu.sync_copy(s_sh, s0_hbm)
        pltpu.sync_copy(agg_sh, agg0_hbm)

    @pl.when(jnp.logical_and(s == 0, c == 1))
    def _():
        pltpu.sync_copy(s_sh, s1_hbm)
        pltpu.sync_copy(agg_sh, agg1_hbm)


def _tc_prepare_body(pref_ref, feat_ref, w_ref, b_ref, o_ref):
    i = pl.program_id(0)

    @pl.when(i < 25)
    def _():
        o_ref[...] = _l2norm(pref_ref[...])

    @pl.when(jnp.logical_and(i >= 25, i < 125))
    def _():
        t = jnp.tanh(jnp.dot(feat_ref[...], w_ref[...],
                             preferred_element_type=_f32) + b_ref[...])
        o_ref[...] = _l2norm(t)

    @pl.when(i >= 125)
    def _():
        o_ref[...] = jnp.zeros_like(o_ref)


def _tc_prepare(preference, features, mlp_w, mlp_b):
    return pl.pallas_call(
        _tc_prepare_body,
        grid=(NP // BM,),
        in_specs=[
            pl.BlockSpec((BM, D), lambda i: (jnp.clip(i, 0, 24), 0)),
            pl.BlockSpec((BM, 128), lambda i: (jnp.clip(i - 25, 0, 99), 0)),
            pl.BlockSpec((128, D), lambda i: (0, 0)),
            pl.BlockSpec((D,), lambda i: (0,)),
        ],
        out_specs=pl.BlockSpec((BM, D), lambda i: (i, 0)),
        out_shape=jax.ShapeDtypeStruct((NP, D), _f32),
    )(preference, features, mlp_w, mlp_b)


def _tc_xw_body(x_ref, w_ref, o_ref):
    o_ref[...] = jnp.dot(x_ref[...], w_ref[...], preferred_element_type=_f32)


def _tc_xw(x_pad, w):
    return pl.pallas_call(
        _tc_xw_body,
        grid=(NP // BM,),
        in_specs=[
            pl.BlockSpec((BM, D), lambda i: (i, 0)),
            pl.BlockSpec((D, D), lambda i: (0, 0)),
        ],
        out_specs=pl.BlockSpec((BM, D), lambda i: (i, 0)),
        out_shape=jax.ShapeDtypeStruct((NP, D), _f32),
    )(x_pad, w)


def _tc_dis_body(d0_ref, d1_ref, o_ref):
    deg = d0_ref[...] + d1_ref[...]
    o_ref[...] = jnp.where(deg > 0, lax.rsqrt(deg), 0.0)


def _tc_dis(deg0, deg1):
    return pl.pallas_call(
        _tc_dis_body,
        grid=(1,),
        in_specs=[pl.BlockSpec((NP,), lambda i: (0,))] * 2,
        out_specs=pl.BlockSpec((NP,), lambda i: (0,)),
        out_shape=jax.ShapeDtypeStruct((NP,), _f32),
    )(deg0, deg1)


def _tc_edge_body(xig_ref, xjg_ref, disg_ref, e_ref, wxj_ref):
    xi = xig_ref[...]
    xj = xjg_ref[...]
    ip = jnp.sum(xi * _leaky(xj), axis=1)
    gate = jax.nn.sigmoid(disg_ref[...] * ip)
    t = ip * gate
    e = jnp.exp(t - 20.0)
    e_ref[...] = e
    wxj_ref[...] = xj * e[:, None]


def _tc_edge(xig, xjg, disg):
    return pl.pallas_call(
        _tc_edge_body,
        grid=(E_PAD // BE,),
        in_specs=[
            pl.BlockSpec((BE, D), lambda i: (i, 0)),
            pl.BlockSpec((BE, D), lambda i: (i, 0)),
            pl.BlockSpec((BE,), lambda i: (i,)),
        ],
        out_specs=[
            pl.BlockSpec((BE,), lambda i: (i,)),
            pl.BlockSpec((BE, D), lambda i: (i, 0)),
        ],
        out_shape=[
            jax.ShapeDtypeStruct((E_PAD,), _f32),
            jax.ShapeDtypeStruct((E_PAD, D), _f32),
        ],
    )(xig, xjg, disg)


def _tc_post_body(a0_ref, a1_ref, s0_ref, s1_ref, cb_ref, x_ref, id_ref,
                  lw_ref, lb_ref, gw_ref, gb_ref, o_ref):
    i = pl.program_id(0)
    sden = s0_ref[...] + s1_ref[...] + 1e-16
    agg = jnp.concatenate([a0_ref[...], a1_ref[...]], axis=1) / sden
    h = _leaky(_l2norm(agg + cb_ref[...]))
    xh = _leaky(jnp.dot(x_ref[...], lw_ref[...],
                        preferred_element_type=_f32) + lb_ref[...])
    xh = xh + id_ref[...]
    xl = _leaky(jnp.dot(h, gw_ref[...],
                        preferred_element_type=_f32) + gb_ref[...] + xh)
    rows = i * BM + lax.broadcasted_iota(_i32, (BM, D), 0)
    o_ref[...] = jnp.where(rows < N, xl, 0.0)


def _tc_post(agg0, agg1, s0, s1, conv_b, x_pad, id_embedding,
             lin_w, lin_b, g_w, g_b):
    return pl.pallas_call(
        _tc_post_body,
        grid=(NP // BM,),
        in_specs=[
            pl.BlockSpec((BM, 32), lambda i: (jnp.clip(i, 0, 124), 0)),
            pl.BlockSpec((BM, 32), lambda i: (jnp.clip(i, 0, 124), 0)),
            pl.BlockSpec((BM, 1), lambda i: (jnp.clip(i, 0, 124), 0)),
            pl.BlockSpec((BM, 1), lambda i: (jnp.clip(i, 0, 124), 0)),
            pl.BlockSpec((D,), lambda i: (0,)),
            pl.BlockSpec((BM, D), lambda i: (i, 0)),
            pl.BlockSpec((BM, D), lambda i: (jnp.clip(i, 0, 124), 0)),
            pl.BlockSpec((D, D), lambda i: (0, 0)),
            pl.BlockSpec((D,), lambda i: (0,)),
            pl.BlockSpec((D, D), lambda i: (0, 0)),
            pl.BlockSpec((D,), lambda i: (0,)),
        ],
        out_specs=pl.BlockSpec((BM, D), lambda i: (i, 0)),
        out_shape=jax.ShapeDtypeStruct((NP, D), _f32),
    )(agg0, agg1, s0.reshape(SNP, 1), s1.reshape(SNP, 1), conv_b, x_pad,
      id_embedding, lin_w, lin_b, g_w, g_b)


def _concat_body(a_ref, b_ref, c_ref, o_ref):
    o_ref[:, 0:64] = a_ref[...]
    o_ref[:, 64:128] = b_ref[...]
    o_ref[:, 128:192] = c_ref[...]


def _tc_concat(x1, x2, x3):
    return pl.pallas_call(
        _concat_body,
        grid=(N // BM,),
        in_specs=[pl.BlockSpec((BM, D), lambda i: (i, 0))] * 3,
        out_specs=pl.BlockSpec((BM, 192), lambda i: (i, 0)),
        out_shape=jax.ShapeDtypeStruct((N, 192), _f32),
    )(x1, x2, x3)


def _gat_layer(x_pad, conv_w, rowm, colm, dis):
    xw = _tc_xw(x_pad, conv_w)
    xjg, xig, disg = _sc_gather(xw, dis, rowm, colm)
    e_arr, wxj = _tc_edge(xig, xjg, disg)
    s0, s1, agg0, agg1 = _sc_scatter(wxj, e_arr, colm)
    return s0, s1, agg0, agg1


def kernel(features, preference, id_embedding, edge_index,
           mlp_w, mlp_b, conv1_w, conv1_b, lin1_w, lin1_b, g1_w, g1_b,
           conv2_w, conv2_b, lin2_w, lin2_b, g2_w, g2_b,
           conv3_w, conv3_b, lin3_w, lin3_b, g3_w, g3_b):
    row = jnp.pad(edge_index[0], (0, E_PAD - E)).reshape(ROWS2, CH)
    col = jnp.pad(edge_index[1], (0, E_PAD - E)).reshape(ROWS2, CH)

    rowm, colm, deg0, deg1 = _sc_deg(row, col)
    dis = _tc_dis(deg0, deg1)

    x = _tc_prepare(preference, features, mlp_w, mlp_b)

    s0, s1, agg0, agg1 = _gat_layer(x, conv1_w, rowm, colm, dis)
    x1 = _tc_post(agg0, agg1, s0, s1, conv1_b, x, id_embedding,
                  lin1_w, lin1_b, g1_w, g1_b)

    s0, s1, agg0, agg1 = _gat_layer(x1, conv2_w, rowm, colm, dis)
    x2 = _tc_post(agg0, agg1, s0, s1, conv2_b, x1, id_embedding,
                  lin2_w, lin2_b, g2_w, g2_b)

    s0, s1, agg0, agg1 = _gat_layer(x2, conv3_w, rowm, colm, dis)
    x3 = _tc_post(agg0, agg1, s0, s1, conv3_b, x2, id_embedding,
                  lin3_w, lin3_b, g3_w, g3_b)

    return _tc_concat(x1, x2, x3)
